# Initial kernel scaffold; baseline (speedup 1.0000x reference)
import jax
import jax.numpy as jnp
from jax.experimental import pallas as pl


def kernel(token_ids, weight):
    raise NotImplementedError("write your pallas kernel here")


# speedup vs baseline: 2.9748x; 2.9748x over previous
import functools

import jax
import jax.numpy as jnp
from jax import lax
from jax.experimental import pallas as pl
from jax.experimental.pallas import tpu as pltpu
from jax.experimental.pallas import tpu_sc as plsc

NUM_CORES = 2
NUM_SUBCORES = 16
NUM_WORKERS = NUM_CORES * NUM_SUBCORES
CHUNK = 128


@functools.partial(jax.jit, static_argnames=())
def _sc_gather(idx3, table):
    nw, n_chunks, chunk = idx3.shape
    d = table.shape[1]
    b_total = nw * n_chunks * chunk
    mesh = plsc.VectorSubcoreMesh(core_axis_name="c", subcore_axis_name="s")

    @functools.partial(
        pl.kernel,
        out_type=jax.ShapeDtypeStruct((b_total, d), table.dtype),
        mesh=mesh,
        scratch_types=[
            pltpu.VMEM((n_chunks, chunk), jnp.int32),
            pltpu.VMEM((chunk, d), table.dtype),
            pltpu.SemaphoreType.DMA,
        ],
    )
    def body(idx_hbm, table_hbm, out_hbm, idx_v, rows_v, sem):
        wid = lax.axis_index("s") * NUM_CORES + lax.axis_index("c")
        pltpu.sync_copy(idx_hbm.at[wid], idx_v)

        def step(j, carry):
            pltpu.async_copy(table_hbm.at[idx_v.at[j]], rows_v, sem).wait()
            base = (wid * n_chunks + j) * chunk
            pltpu.sync_copy(rows_v, out_hbm.at[pl.ds(base, chunk)])
            return carry

        lax.fori_loop(0, n_chunks, step, 0)

    return body(idx3, table)


def kernel(token_ids, weight):
    b, s = token_ids.shape
    d = weight.shape[1]
    idx = token_ids.reshape(-1).astype(jnp.int32)
    idx3 = idx.reshape(NUM_WORKERS, -1, CHUNK)
    out = _sc_gather(idx3, weight)
    return out.reshape(b, s, d)


# speedup vs baseline: 3.3459x; 1.1248x over previous
import functools

import jax
import jax.numpy as jnp
from jax import lax
from jax.experimental import pallas as pl
from jax.experimental.pallas import tpu as pltpu
from jax.experimental.pallas import tpu_sc as plsc

NUM_CORES = 2
NUM_SUBCORES = 16
NUM_WORKERS = NUM_CORES * NUM_SUBCORES
CHUNK = 128


@functools.partial(jax.jit, static_argnames=())
def _sc_gather(idx3, table):
    nw, n_chunks, chunk = idx3.shape
    d = table.shape[1]
    b_total = nw * n_chunks * chunk
    mesh = plsc.VectorSubcoreMesh(core_axis_name="c", subcore_axis_name="s")

    @functools.partial(
        pl.kernel,
        out_type=jax.ShapeDtypeStruct((b_total, d), table.dtype),
        mesh=mesh,
        scratch_types=[
            pltpu.VMEM((n_chunks, chunk), jnp.int32),
            pltpu.VMEM((2, chunk, d), table.dtype),
            pltpu.SemaphoreType.DMA,
            pltpu.SemaphoreType.DMA,
        ],
    )
    def body(idx_hbm, table_hbm, out_hbm, idx_v, rows_v, g0, g1):
        wid = lax.axis_index("s") * NUM_CORES + lax.axis_index("c")
        pltpu.sync_copy(idx_hbm.at[wid], idx_v)
        gsems = (g0, g1)

        def start_gather(j, b):
            pltpu.async_copy(table_hbm.at[idx_v.at[j]], rows_v.at[b], gsems[b])

        def wait_gather(j, b):
            pltpu.make_async_copy(
                table_hbm.at[idx_v.at[j]], rows_v.at[b], gsems[b]
            ).wait()

        start_gather(0, 0)

        def outer(i, carry):
            for b in range(2):
                j = 2 * i + b

                @pl.when(j + 1 < n_chunks)
                def _():
                    start_gather(j + 1, (b + 1) % 2)

                wait_gather(j, b)
                base = (wid * n_chunks + j) * chunk
                pltpu.sync_copy(rows_v.at[b], out_hbm.at[pl.ds(base, chunk)])
            return carry

        lax.fori_loop(0, n_chunks // 2, outer, 0)

    return body(idx3, table)


def kernel(token_ids, weight):
    b, s = token_ids.shape
    d = weight.shape[1]
    idx = token_ids.reshape(-1).astype(jnp.int32)
    idx3 = idx.reshape(NUM_WORKERS, -1, CHUNK)
    out = _sc_gather(idx3, weight)
    return out.reshape(b, s, d)
